# baseline (speedup 1.0000x reference)
import functools

import jax
import jax.numpy as jnp
from jax import lax
from jax.experimental import pallas as pl
from jax.experimental.pallas import tpu as pltpu
from jax.experimental.pallas import tpu_sc as plsc

_BINS = 15
_N = 16384
_C = 1000

_SC_N = 8192
_NW = 32
_RPW = _SC_N // _NW
_B = 16
_T = _RPW // _B

_R = 2048
_TC_OFF = _SC_N // _R

_EDGES = [
    0.0, 0.06666667014360428, 0.13333334028720856, 0.20000001788139343,
    0.2666666805744171, 0.3333333432674408, 0.40000003576278687,
    0.46666669845581055, 0.5333333611488342, 0.6000000238418579,
    0.6666666865348816, 0.7333333492279053, 0.8000000715255737,
    0.8666667342185974, 0.9333333969116211, 1.0,
]


def _sc_body(logits_hbm, labels_hbm, out_hbm, buf, lab, cnt, csum, asum, sem):
    w = lax.axis_index("s") * 2 + lax.axis_index("c")
    base = w * _RPW
    row_iota = lax.broadcasted_iota(jnp.int32, (_B,), 0)

    for b in range(_BINS):
        cnt[b, :] = jnp.zeros((_B,), jnp.float32)
        csum[b, :] = jnp.zeros((_B,), jnp.float32)
        asum[b, :] = jnp.zeros((_B,), jnp.float32)

    pltpu.async_copy(
        logits_hbm.at[pl.ds(base, _B), :], buf.at[0], sem.at[0])
    pltpu.async_copy(
        logits_hbm.at[pl.ds(base + _B, _B), :], buf.at[1], sem.at[1])

    def process(slot, row0):
        pltpu.make_async_copy(
            logits_hbm.at[pl.ds(0, _B), :], buf.at[slot], sem.at[slot]).wait()

        def jbody(j, carry):
            m, s = carry
            colj = jnp.full((_B,), 0, jnp.int32) + j
            v = plsc.load_gather(buf.at[slot], [row_iota, colj])
            return jnp.maximum(m, v), s + jnp.exp(v)

        m, s = lax.fori_loop(
            0, _C, jbody,
            (jnp.full((_B,), -jnp.inf, jnp.float32),
             jnp.zeros((_B,), jnp.float32)))

        pltpu.sync_copy(labels_hbm.at[pl.ds(row0, _B)], lab)
        xl = plsc.load_gather(buf.at[slot], [row_iota, lab[...]])
        correct = jnp.where(xl == m, 1.0, 0.0).astype(jnp.float32)
        p = jnp.exp(m) / s

        bidx = jnp.zeros((_B,), jnp.int32)
        for e in _EDGES[1:]:
            bidx = bidx + jnp.where(p > e, 1, 0).astype(jnp.int32)
        for b in range(_BINS):
            mask = bidx == b
            cnt[b, :] = cnt[b, :] + jnp.where(mask, 1.0, 0.0)
            csum[b, :] = csum[b, :] + jnp.where(mask, p, 0.0)
            asum[b, :] = asum[b, :] + jnp.where(mask, correct, 0.0)

    def obody(g, carry):
        for slot in range(2):
            t = 2 * g + slot
            row0 = base + t * _B
            process(slot, row0)
            nxt = t + 2

            @pl.when(nxt < _T)
            def _():
                pltpu.async_copy(
                    logits_hbm.at[pl.ds(base + nxt * _B, _B), :],
                    buf.at[slot], sem.at[slot])
        return carry

    lax.fori_loop(0, _T // 2, obody, 0)

    pltpu.sync_copy(cnt, out_hbm.at[w, 0])
    pltpu.sync_copy(csum, out_hbm.at[w, 1])
    pltpu.sync_copy(asum, out_hbm.at[w, 2])


def _sc_partials(logits, labels):
    mesh = plsc.VectorSubcoreMesh(core_axis_name="c", subcore_axis_name="s")
    kern = functools.partial(
        pl.kernel,
        out_type=jax.ShapeDtypeStruct((_NW, 3, _BINS, _B), jnp.float32),
        mesh=mesh,
        scratch_types=[
            pltpu.VMEM((2, _B, _C), jnp.float32),
            pltpu.VMEM((_B,), jnp.int32),
            pltpu.VMEM((_BINS, _B), jnp.float32),
            pltpu.VMEM((_BINS, _B), jnp.float32),
            pltpu.VMEM((_BINS, _B), jnp.float32),
            pltpu.SemaphoreType.DMA((2,)),
        ],
        compiler_params=pltpu.CompilerParams(needs_layout_passes=False),
    )(_sc_body)
    return kern(logits, labels)


def _tc_block(logits_ref, labels_ref, cnt_ref, csum_ref, asum_ref):
    step = pl.program_id(0)

    @pl.when(step == 0)
    def _init():
        cnt_ref[...] = jnp.zeros_like(cnt_ref)
        csum_ref[...] = jnp.zeros_like(csum_ref)
        asum_ref[...] = jnp.zeros_like(asum_ref)

    x = logits_ref[...]
    labels = labels_ref[0, 0, :]

    m = jnp.max(x, axis=-1, keepdims=True)
    s = jnp.sum(jnp.exp(x - m), axis=-1)
    p = 1.0 / s
    col = jax.lax.broadcasted_iota(jnp.int32, x.shape, 1)
    pred = jnp.min(jnp.where(x == m, col, _C), axis=-1)
    correct = (pred == labels).astype(jnp.float32)

    lane = jax.lax.broadcasted_iota(jnp.int32, (1, 128), 1)
    lo_row = jnp.full((1, 128), 2.0, dtype=jnp.float32)
    hi_row = jnp.full((1, 128), 3.0, dtype=jnp.float32)
    for i in range(_BINS):
        lo_row = jnp.where(lane == i, _EDGES[i], lo_row)
        hi_row = jnp.where(lane == i, _EDGES[i + 1], hi_row)
    pd = p[:, None]
    onehot = ((pd > lo_row) & (pd <= hi_row)).astype(jnp.float32)

    cnt_ref[...] += jnp.sum(onehot, axis=0, keepdims=True)
    csum_ref[...] += jnp.sum(onehot * pd, axis=0, keepdims=True)
    asum_ref[...] += jnp.sum(onehot * correct[:, None], axis=0, keepdims=True)


def _combine(sc_ref, cnt_ref, csum_ref, asum_ref, ece_ref):
    red = jnp.sum(jnp.sum(sc_ref[...], axis=0), axis=-1)
    cnt = red[0:1, :] + cnt_ref[0:1, 0:_BINS]
    csum = red[1:2, :] + csum_ref[0:1, 0:_BINS]
    asum = red[2:3, :] + asum_ref[0:1, 0:_BINS]
    safe = jnp.where(cnt > 0, cnt, 1.0)
    e = jnp.where(cnt > 0, csum / safe - asum / safe, 0.0)
    ece_ref[...] = jnp.sum(jnp.abs(e) * (cnt / _N)).reshape(1, 1)


@jax.jit
def _ece(logits, labels):
    labels = labels.astype(jnp.int32)
    sc_out = _sc_partials(logits, labels)

    grid = (_N - _SC_N) // _R
    labels3 = labels.reshape(_N // _R, 1, _R)
    acc = jax.ShapeDtypeStruct((1, 128), jnp.float32)
    cnt, csum, asum = pl.pallas_call(
        _tc_block,
        grid=(grid,),
        in_specs=[
            pl.BlockSpec((_R, _C), lambda i: (i + _TC_OFF, 0)),
            pl.BlockSpec((1, 1, _R), lambda i: (i + _TC_OFF, 0, 0)),
        ],
        out_specs=[
            pl.BlockSpec((1, 128), lambda i: (0, 0)),
            pl.BlockSpec((1, 128), lambda i: (0, 0)),
            pl.BlockSpec((1, 128), lambda i: (0, 0)),
        ],
        out_shape=[acc, acc, acc],
    )(logits, labels3)

    ece = pl.pallas_call(
        _combine,
        out_shape=jax.ShapeDtypeStruct((1, 1), jnp.float32),
    )(sc_out, cnt, csum, asum)
    return ece[0, 0]


def kernel(logits, labels):
    return _ece(logits, labels)


# speedup vs baseline: 1.4284x; 1.4284x over previous
import functools

import jax
import jax.numpy as jnp
from jax import lax
from jax.experimental import pallas as pl
from jax.experimental.pallas import tpu as pltpu
from jax.experimental.pallas import tpu_sc as plsc

_BINS = 15
_N = 16384
_C = 1000

_SC_N = 8192
_NW = 32
_RPW = _SC_N // _NW
_B = 16
_T = _RPW // _B

_R = 2048
_TC_OFF = _SC_N // _R

_EDGES = [
    0.0, 0.06666667014360428, 0.13333334028720856, 0.20000001788139343,
    0.2666666805744171, 0.3333333432674408, 0.40000003576278687,
    0.46666669845581055, 0.5333333611488342, 0.6000000238418579,
    0.6666666865348816, 0.7333333492279053, 0.8000000715255737,
    0.8666667342185974, 0.9333333969116211, 1.0,
]


def _sc_body(logits_hbm, labels_hbm, out_hbm, buf, lab, cnt, csum, asum, sem):
    w = lax.axis_index("s") * 2 + lax.axis_index("c")
    base = w * _RPW
    row_iota = lax.broadcasted_iota(jnp.int32, (_B,), 0)

    for b in range(_BINS):
        cnt[b, :] = jnp.zeros((_B,), jnp.float32)
        csum[b, :] = jnp.zeros((_B,), jnp.float32)
        asum[b, :] = jnp.zeros((_B,), jnp.float32)

    pltpu.async_copy(
        logits_hbm.at[pl.ds(base, _B), :], buf.at[0], sem.at[0])
    pltpu.async_copy(
        logits_hbm.at[pl.ds(base + _B, _B), :], buf.at[1], sem.at[1])

    neg_inf = jnp.full((_B,), -jnp.inf, jnp.float32)
    zeros = jnp.zeros((_B,), jnp.float32)

    def process(slot, row0):
        pltpu.make_async_copy(
            logits_hbm.at[pl.ds(0, _B), :], buf.at[slot], sem.at[slot]).wait()

        def jbody(c, carry):
            cvec, ms, ss = carry
            ms2, ss2 = list(ms), list(ss)
            for kk in range(8):
                v = plsc.load_gather(buf.at[slot], [row_iota, cvec + kk])
                k = kk % 4
                ms2[k] = jnp.maximum(ms2[k], v)
                ss2[k] = ss2[k] + jnp.exp(v)
            return cvec + 8, tuple(ms2), tuple(ss2)

        _, ms, ss = lax.fori_loop(
            0, _C // 8, jbody,
            (jnp.zeros((_B,), jnp.int32), (neg_inf,) * 4, (zeros,) * 4))
        m = jnp.maximum(jnp.maximum(ms[0], ms[1]), jnp.maximum(ms[2], ms[3]))
        s = (ss[0] + ss[1]) + (ss[2] + ss[3])

        pltpu.sync_copy(labels_hbm.at[pl.ds(row0, _B)], lab)
        xl = plsc.load_gather(buf.at[slot], [row_iota, lab[...]])
        correct = jnp.where(xl == m, 1.0, 0.0).astype(jnp.float32)
        p = jnp.exp(m) / s

        bidx = jnp.zeros((_B,), jnp.int32)
        for e in _EDGES[1:]:
            bidx = bidx + jnp.where(p > e, 1, 0).astype(jnp.int32)
        for b in range(_BINS):
            mask = bidx == b
            cnt[b, :] = cnt[b, :] + jnp.where(mask, 1.0, 0.0)
            csum[b, :] = csum[b, :] + jnp.where(mask, p, 0.0)
            asum[b, :] = asum[b, :] + jnp.where(mask, correct, 0.0)

    def obody(g, carry):
        for slot in range(2):
            t = 2 * g + slot
            row0 = base + t * _B
            process(slot, row0)
            nxt = t + 2

            @pl.when(nxt < _T)
            def _():
                pltpu.async_copy(
                    logits_hbm.at[pl.ds(base + nxt * _B, _B), :],
                    buf.at[slot], sem.at[slot])
        return carry

    lax.fori_loop(0, _T // 2, obody, 0)

    pltpu.sync_copy(cnt, out_hbm.at[w, 0])
    pltpu.sync_copy(csum, out_hbm.at[w, 1])
    pltpu.sync_copy(asum, out_hbm.at[w, 2])


def _sc_partials(logits, labels):
    mesh = plsc.VectorSubcoreMesh(core_axis_name="c", subcore_axis_name="s")
    kern = functools.partial(
        pl.kernel,
        out_type=jax.ShapeDtypeStruct((_NW, 3, _BINS, _B), jnp.float32),
        mesh=mesh,
        scratch_types=[
            pltpu.VMEM((2, _B, _C), jnp.float32),
            pltpu.VMEM((_B,), jnp.int32),
            pltpu.VMEM((_BINS, _B), jnp.float32),
            pltpu.VMEM((_BINS, _B), jnp.float32),
            pltpu.VMEM((_BINS, _B), jnp.float32),
            pltpu.SemaphoreType.DMA((2,)),
        ],
        compiler_params=pltpu.CompilerParams(needs_layout_passes=False),
    )(_sc_body)
    return kern(logits, labels)


def _tc_block(logits_ref, labels_ref, cnt_ref, csum_ref, asum_ref):
    step = pl.program_id(0)

    @pl.when(step == 0)
    def _init():
        cnt_ref[...] = jnp.zeros_like(cnt_ref)
        csum_ref[...] = jnp.zeros_like(csum_ref)
        asum_ref[...] = jnp.zeros_like(asum_ref)

    x = logits_ref[...]
    labels = labels_ref[0, 0, :]

    m = jnp.max(x, axis=-1, keepdims=True)
    s = jnp.sum(jnp.exp(x - m), axis=-1)
    p = 1.0 / s
    col = jax.lax.broadcasted_iota(jnp.int32, x.shape, 1)
    pred = jnp.min(jnp.where(x == m, col, _C), axis=-1)
    correct = (pred == labels).astype(jnp.float32)

    lane = jax.lax.broadcasted_iota(jnp.int32, (1, 128), 1)
    lo_row = jnp.full((1, 128), 2.0, dtype=jnp.float32)
    hi_row = jnp.full((1, 128), 3.0, dtype=jnp.float32)
    for i in range(_BINS):
        lo_row = jnp.where(lane == i, _EDGES[i], lo_row)
        hi_row = jnp.where(lane == i, _EDGES[i + 1], hi_row)
    pd = p[:, None]
    onehot = ((pd > lo_row) & (pd <= hi_row)).astype(jnp.float32)

    cnt_ref[...] += jnp.sum(onehot, axis=0, keepdims=True)
    csum_ref[...] += jnp.sum(onehot * pd, axis=0, keepdims=True)
    asum_ref[...] += jnp.sum(onehot * correct[:, None], axis=0, keepdims=True)


def _combine(sc_ref, cnt_ref, csum_ref, asum_ref, ece_ref):
    red = jnp.sum(jnp.sum(sc_ref[...], axis=0), axis=-1)
    cnt = red[0:1, :] + cnt_ref[0:1, 0:_BINS]
    csum = red[1:2, :] + csum_ref[0:1, 0:_BINS]
    asum = red[2:3, :] + asum_ref[0:1, 0:_BINS]
    safe = jnp.where(cnt > 0, cnt, 1.0)
    e = jnp.where(cnt > 0, csum / safe - asum / safe, 0.0)
    ece_ref[...] = jnp.sum(jnp.abs(e) * (cnt / _N)).reshape(1, 1)


@jax.jit
def _ece(logits, labels):
    labels = labels.astype(jnp.int32)
    sc_out = _sc_partials(logits, labels)

    grid = (_N - _SC_N) // _R
    labels3 = labels.reshape(_N // _R, 1, _R)
    acc = jax.ShapeDtypeStruct((1, 128), jnp.float32)
    cnt, csum, asum = pl.pallas_call(
        _tc_block,
        grid=(grid,),
        in_specs=[
            pl.BlockSpec((_R, _C), lambda i: (i + _TC_OFF, 0)),
            pl.BlockSpec((1, 1, _R), lambda i: (i + _TC_OFF, 0, 0)),
        ],
        out_specs=[
            pl.BlockSpec((1, 128), lambda i: (0, 0)),
            pl.BlockSpec((1, 128), lambda i: (0, 0)),
            pl.BlockSpec((1, 128), lambda i: (0, 0)),
        ],
        out_shape=[acc, acc, acc],
    )(logits, labels3)

    ece = pl.pallas_call(
        _combine,
        out_shape=jax.ShapeDtypeStruct((1, 1), jnp.float32),
    )(sc_out, cnt, csum, asum)
    return ece[0, 0]


def kernel(logits, labels):
    return _ece(logits, labels)


# speedup vs baseline: 2.2371x; 1.5662x over previous
import functools

import jax
import jax.numpy as jnp
from jax import lax
from jax.experimental import pallas as pl
from jax.experimental.pallas import tpu as pltpu
from jax.experimental.pallas import tpu_sc as plsc

_BINS = 15
_N = 16384
_C = 1000

_SC_N = 8192
_NW = 32
_RPW = _SC_N // _NW
_B = 16
_T = _RPW // _B

_R = 2048
_TC_OFF = _SC_N // _R

_EDGES = [
    0.0, 0.06666667014360428, 0.13333334028720856, 0.20000001788139343,
    0.2666666805744171, 0.3333333432674408, 0.40000003576278687,
    0.46666669845581055, 0.5333333611488342, 0.6000000238418579,
    0.6666666865348816, 0.7333333492279053, 0.8000000715255737,
    0.8666667342185974, 0.9333333969116211, 1.0,
]


def _sc_body(logits_hbm, labels_hbm, out_hbm, buf, lab, cnt, csum, asum, sem):
    w = lax.axis_index("s") * 2 + lax.axis_index("c")
    base = w * _RPW
    row_iota = lax.broadcasted_iota(jnp.int32, (_B,), 0)

    for b in range(_BINS):
        cnt[b, :] = jnp.zeros((_B,), jnp.float32)
        csum[b, :] = jnp.zeros((_B,), jnp.float32)
        asum[b, :] = jnp.zeros((_B,), jnp.float32)

    pltpu.async_copy(
        logits_hbm.at[pl.ds(base, _B), :], buf.at[0], sem.at[0])
    pltpu.async_copy(
        logits_hbm.at[pl.ds(base + _B, _B), :], buf.at[1], sem.at[1])

    neg_inf = jnp.full((_B,), -jnp.inf, jnp.float32)
    zeros = jnp.zeros((_B,), jnp.float32)

    def process(slot, row0):
        pltpu.make_async_copy(
            logits_hbm.at[pl.ds(0, _B), :], buf.at[slot], sem.at[slot]).wait()

        def jbody(c, carry):
            cvec, ms, ss = carry
            ms2, ss2 = list(ms), list(ss)
            for kk in range(8):
                v = plsc.load_gather(buf.at[slot], [row_iota, cvec + kk])
                k = kk % 4
                ms2[k] = jnp.maximum(ms2[k], v)
                ss2[k] = ss2[k] + jnp.exp(v)
            return cvec + 8, tuple(ms2), tuple(ss2)

        _, ms, ss = lax.fori_loop(
            0, 123, jbody,
            (row_iota, (neg_inf,) * 4, (zeros,) * 4))
        ms2, ss2 = list(ms), list(ss)
        for kk in range(16):
            raw = row_iota + (984 + kk)
            col = jnp.where(raw >= _C, raw - _C, raw)
            v = plsc.load_gather(buf.at[slot], [row_iota, col])
            k = kk % 4
            ms2[k] = jnp.maximum(ms2[k], v)
            ss2[k] = ss2[k] + jnp.exp(v)
        ms, ss = ms2, ss2
        m = jnp.maximum(jnp.maximum(ms[0], ms[1]), jnp.maximum(ms[2], ms[3]))
        s = (ss[0] + ss[1]) + (ss[2] + ss[3])

        pltpu.sync_copy(labels_hbm.at[pl.ds(row0, _B)], lab)
        xl = plsc.load_gather(buf.at[slot], [row_iota, lab[...]])
        correct = jnp.where(xl == m, 1.0, 0.0).astype(jnp.float32)
        p = jnp.exp(m) / s

        bidx = jnp.zeros((_B,), jnp.int32)
        for e in _EDGES[1:]:
            bidx = bidx + jnp.where(p > e, 1, 0).astype(jnp.int32)
        for b in range(_BINS):
            mask = bidx == b
            cnt[b, :] = cnt[b, :] + jnp.where(mask, 1.0, 0.0)
            csum[b, :] = csum[b, :] + jnp.where(mask, p, 0.0)
            asum[b, :] = asum[b, :] + jnp.where(mask, correct, 0.0)

    def obody(g, carry):
        for slot in range(2):
            t = 2 * g + slot
            row0 = base + t * _B
            process(slot, row0)
            nxt = t + 2

            @pl.when(nxt < _T)
            def _():
                pltpu.async_copy(
                    logits_hbm.at[pl.ds(base + nxt * _B, _B), :],
                    buf.at[slot], sem.at[slot])
        return carry

    lax.fori_loop(0, _T // 2, obody, 0)

    pltpu.sync_copy(cnt, out_hbm.at[w, 0])
    pltpu.sync_copy(csum, out_hbm.at[w, 1])
    pltpu.sync_copy(asum, out_hbm.at[w, 2])


def _sc_partials(logits, labels):
    mesh = plsc.VectorSubcoreMesh(core_axis_name="c", subcore_axis_name="s")
    kern = functools.partial(
        pl.kernel,
        out_type=jax.ShapeDtypeStruct((_NW, 3, _BINS, _B), jnp.float32),
        mesh=mesh,
        scratch_types=[
            pltpu.VMEM((2, _B, _C), jnp.float32),
            pltpu.VMEM((_B,), jnp.int32),
            pltpu.VMEM((_BINS, _B), jnp.float32),
            pltpu.VMEM((_BINS, _B), jnp.float32),
            pltpu.VMEM((_BINS, _B), jnp.float32),
            pltpu.SemaphoreType.DMA((2,)),
        ],
        compiler_params=pltpu.CompilerParams(needs_layout_passes=False),
    )(_sc_body)
    return kern(logits, labels)


def _tc_block(logits_ref, labels_ref, cnt_ref, csum_ref, asum_ref):
    step = pl.program_id(0)

    @pl.when(step == 0)
    def _init():
        cnt_ref[...] = jnp.zeros_like(cnt_ref)
        csum_ref[...] = jnp.zeros_like(csum_ref)
        asum_ref[...] = jnp.zeros_like(asum_ref)

    x = logits_ref[...]
    labels = labels_ref[0, 0, :]

    m = jnp.max(x, axis=-1, keepdims=True)
    s = jnp.sum(jnp.exp(x - m), axis=-1)
    p = 1.0 / s
    col = jax.lax.broadcasted_iota(jnp.int32, x.shape, 1)
    pred = jnp.min(jnp.where(x == m, col, _C), axis=-1)
    correct = (pred == labels).astype(jnp.float32)

    lane = jax.lax.broadcasted_iota(jnp.int32, (1, 128), 1)
    lo_row = jnp.full((1, 128), 2.0, dtype=jnp.float32)
    hi_row = jnp.full((1, 128), 3.0, dtype=jnp.float32)
    for i in range(_BINS):
        lo_row = jnp.where(lane == i, _EDGES[i], lo_row)
        hi_row = jnp.where(lane == i, _EDGES[i + 1], hi_row)
    pd = p[:, None]
    onehot = ((pd > lo_row) & (pd <= hi_row)).astype(jnp.float32)

    cnt_ref[...] += jnp.sum(onehot, axis=0, keepdims=True)
    csum_ref[...] += jnp.sum(onehot * pd, axis=0, keepdims=True)
    asum_ref[...] += jnp.sum(onehot * correct[:, None], axis=0, keepdims=True)


def _combine(sc_ref, cnt_ref, csum_ref, asum_ref, ece_ref):
    red = jnp.sum(jnp.sum(sc_ref[...], axis=0), axis=-1)
    cnt = red[0:1, :] + cnt_ref[0:1, 0:_BINS]
    csum = red[1:2, :] + csum_ref[0:1, 0:_BINS]
    asum = red[2:3, :] + asum_ref[0:1, 0:_BINS]
    safe = jnp.where(cnt > 0, cnt, 1.0)
    e = jnp.where(cnt > 0, csum / safe - asum / safe, 0.0)
    ece_ref[...] = jnp.sum(jnp.abs(e) * (cnt / _N)).reshape(1, 1)


@jax.jit
def _ece(logits, labels):
    labels = labels.astype(jnp.int32)
    sc_out = _sc_partials(logits, labels)

    grid = (_N - _SC_N) // _R
    labels3 = labels.reshape(_N // _R, 1, _R)
    acc = jax.ShapeDtypeStruct((1, 128), jnp.float32)
    cnt, csum, asum = pl.pallas_call(
        _tc_block,
        grid=(grid,),
        in_specs=[
            pl.BlockSpec((_R, _C), lambda i: (i + _TC_OFF, 0)),
            pl.BlockSpec((1, 1, _R), lambda i: (i + _TC_OFF, 0, 0)),
        ],
        out_specs=[
            pl.BlockSpec((1, 128), lambda i: (0, 0)),
            pl.BlockSpec((1, 128), lambda i: (0, 0)),
            pl.BlockSpec((1, 128), lambda i: (0, 0)),
        ],
        out_shape=[acc, acc, acc],
    )(logits, labels3)

    ece = pl.pallas_call(
        _combine,
        out_shape=jax.ShapeDtypeStruct((1, 1), jnp.float32),
    )(sc_out, cnt, csum, asum)
    return ece[0, 0]


def kernel(logits, labels):
    return _ece(logits, labels)


# speedup vs baseline: 2.6250x; 1.1734x over previous
import functools

import jax
import jax.numpy as jnp
from jax import lax
from jax.experimental import pallas as pl
from jax.experimental.pallas import tpu as pltpu
from jax.experimental.pallas import tpu_sc as plsc

_BINS = 15
_N = 16384
_C = 1000

_SC_N = 4096
_NW = 32
_RPW = _SC_N // _NW
_B = 16
_T = _RPW // _B

_R = 2048
_TC_OFF = _SC_N // _R

_EDGES = [
    0.0, 0.06666667014360428, 0.13333334028720856, 0.20000001788139343,
    0.2666666805744171, 0.3333333432674408, 0.40000003576278687,
    0.46666669845581055, 0.5333333611488342, 0.6000000238418579,
    0.6666666865348816, 0.7333333492279053, 0.8000000715255737,
    0.8666667342185974, 0.9333333969116211, 1.0,
]


def _sc_body(logits_hbm, labels_hbm, out_hbm, buf, lab, cnt, csum, asum, sem):
    w = lax.axis_index("s") * 2 + lax.axis_index("c")
    base = w * _RPW
    row_iota = lax.broadcasted_iota(jnp.int32, (_B,), 0)

    for b in range(_BINS):
        cnt[b, :] = jnp.zeros((_B,), jnp.float32)
        csum[b, :] = jnp.zeros((_B,), jnp.float32)
        asum[b, :] = jnp.zeros((_B,), jnp.float32)

    pltpu.async_copy(
        logits_hbm.at[pl.ds(base, _B), :], buf.at[0], sem.at[0])
    pltpu.async_copy(
        logits_hbm.at[pl.ds(base + _B, _B), :], buf.at[1], sem.at[1])

    neg_inf = jnp.full((_B,), -jnp.inf, jnp.float32)
    zeros = jnp.zeros((_B,), jnp.float32)

    def process(slot, row0):
        pltpu.make_async_copy(
            logits_hbm.at[pl.ds(0, _B), :], buf.at[slot], sem.at[slot]).wait()

        def jbody(c, carry):
            cvec, ms, ss = carry
            ms2, ss2 = list(ms), list(ss)
            for kk in range(8):
                v = plsc.load_gather(buf.at[slot], [row_iota, cvec + kk])
                k = kk % 4
                ms2[k] = jnp.maximum(ms2[k], v)
                ss2[k] = ss2[k] + jnp.exp(v)
            return cvec + 8, tuple(ms2), tuple(ss2)

        _, ms, ss = lax.fori_loop(
            0, 123, jbody,
            (row_iota, (neg_inf,) * 4, (zeros,) * 4))
        ms2, ss2 = list(ms), list(ss)
        for kk in range(16):
            raw = row_iota + (984 + kk)
            col = jnp.where(raw >= _C, raw - _C, raw)
            v = plsc.load_gather(buf.at[slot], [row_iota, col])
            k = kk % 4
            ms2[k] = jnp.maximum(ms2[k], v)
            ss2[k] = ss2[k] + jnp.exp(v)
        ms, ss = ms2, ss2
        m = jnp.maximum(jnp.maximum(ms[0], ms[1]), jnp.maximum(ms[2], ms[3]))
        s = (ss[0] + ss[1]) + (ss[2] + ss[3])

        pltpu.sync_copy(labels_hbm.at[pl.ds(row0, _B)], lab)
        xl = plsc.load_gather(buf.at[slot], [row_iota, lab[...]])
        correct = jnp.where(xl == m, 1.0, 0.0).astype(jnp.float32)
        p = jnp.exp(m) / s

        bidx = jnp.zeros((_B,), jnp.int32)
        for e in _EDGES[1:]:
            bidx = bidx + jnp.where(p > e, 1, 0).astype(jnp.int32)
        for b in range(_BINS):
            mask = bidx == b
            cnt[b, :] = cnt[b, :] + jnp.where(mask, 1.0, 0.0)
            csum[b, :] = csum[b, :] + jnp.where(mask, p, 0.0)
            asum[b, :] = asum[b, :] + jnp.where(mask, correct, 0.0)

    def obody(g, carry):
        for slot in range(2):
            t = 2 * g + slot
            row0 = base + t * _B
            process(slot, row0)
            nxt = t + 2

            @pl.when(nxt < _T)
            def _():
                pltpu.async_copy(
                    logits_hbm.at[pl.ds(base + nxt * _B, _B), :],
                    buf.at[slot], sem.at[slot])
        return carry

    lax.fori_loop(0, _T // 2, obody, 0)

    pltpu.sync_copy(cnt, out_hbm.at[w, 0])
    pltpu.sync_copy(csum, out_hbm.at[w, 1])
    pltpu.sync_copy(asum, out_hbm.at[w, 2])


def _sc_partials(logits, labels):
    mesh = plsc.VectorSubcoreMesh(core_axis_name="c", subcore_axis_name="s")
    kern = functools.partial(
        pl.kernel,
        out_type=jax.ShapeDtypeStruct((_NW, 3, _BINS, _B), jnp.float32),
        mesh=mesh,
        scratch_types=[
            pltpu.VMEM((2, _B, _C), jnp.float32),
            pltpu.VMEM((_B,), jnp.int32),
            pltpu.VMEM((_BINS, _B), jnp.float32),
            pltpu.VMEM((_BINS, _B), jnp.float32),
            pltpu.VMEM((_BINS, _B), jnp.float32),
            pltpu.SemaphoreType.DMA((2,)),
        ],
        compiler_params=pltpu.CompilerParams(needs_layout_passes=False),
    )(_sc_body)
    return kern(logits, labels)


def _tc_block(logits_ref, labels_ref, cnt_ref, csum_ref, asum_ref):
    step = pl.program_id(0)

    @pl.when(step == 0)
    def _init():
        cnt_ref[...] = jnp.zeros_like(cnt_ref)
        csum_ref[...] = jnp.zeros_like(csum_ref)
        asum_ref[...] = jnp.zeros_like(asum_ref)

    x = logits_ref[...]
    labels = labels_ref[0, 0, :]

    m = jnp.max(x, axis=-1, keepdims=True)
    s = jnp.sum(jnp.exp(x - m), axis=-1)
    p = 1.0 / s
    col = jax.lax.broadcasted_iota(jnp.int32, x.shape, 1)
    pred = jnp.min(jnp.where(x == m, col, _C), axis=-1)
    correct = (pred == labels).astype(jnp.float32)

    lane = jax.lax.broadcasted_iota(jnp.int32, (1, 128), 1)
    lo_row = jnp.full((1, 128), 2.0, dtype=jnp.float32)
    hi_row = jnp.full((1, 128), 3.0, dtype=jnp.float32)
    for i in range(_BINS):
        lo_row = jnp.where(lane == i, _EDGES[i], lo_row)
        hi_row = jnp.where(lane == i, _EDGES[i + 1], hi_row)
    pd = p[:, None]
    onehot = ((pd > lo_row) & (pd <= hi_row)).astype(jnp.float32)

    cnt_ref[...] += jnp.sum(onehot, axis=0, keepdims=True)
    csum_ref[...] += jnp.sum(onehot * pd, axis=0, keepdims=True)
    asum_ref[...] += jnp.sum(onehot * correct[:, None], axis=0, keepdims=True)


def _combine(sc_ref, cnt_ref, csum_ref, asum_ref, ece_ref):
    red = jnp.sum(jnp.sum(sc_ref[...], axis=0), axis=-1)
    cnt = red[0:1, :] + cnt_ref[0:1, 0:_BINS]
    csum = red[1:2, :] + csum_ref[0:1, 0:_BINS]
    asum = red[2:3, :] + asum_ref[0:1, 0:_BINS]
    safe = jnp.where(cnt > 0, cnt, 1.0)
    e = jnp.where(cnt > 0, csum / safe - asum / safe, 0.0)
    ece_ref[...] = jnp.sum(jnp.abs(e) * (cnt / _N)).reshape(1, 1)


@jax.jit
def _ece(logits, labels):
    labels = labels.astype(jnp.int32)
    sc_out = _sc_partials(logits, labels)

    grid = (_N - _SC_N) // _R
    labels3 = labels.reshape(_N // _R, 1, _R)
    acc = jax.ShapeDtypeStruct((1, 128), jnp.float32)
    cnt, csum, asum = pl.pallas_call(
        _tc_block,
        grid=(grid,),
        in_specs=[
            pl.BlockSpec((_R, _C), lambda i: (i + _TC_OFF, 0)),
            pl.BlockSpec((1, 1, _R), lambda i: (i + _TC_OFF, 0, 0)),
        ],
        out_specs=[
            pl.BlockSpec((1, 128), lambda i: (0, 0)),
            pl.BlockSpec((1, 128), lambda i: (0, 0)),
            pl.BlockSpec((1, 128), lambda i: (0, 0)),
        ],
        out_shape=[acc, acc, acc],
    )(logits, labels3)

    ece = pl.pallas_call(
        _combine,
        out_shape=jax.ShapeDtypeStruct((1, 1), jnp.float32),
    )(sc_out, cnt, csum, asum)
    return ece[0, 0]


def kernel(logits, labels):
    return _ece(logits, labels)


# speedup vs baseline: 2.6830x; 1.0221x over previous
import functools

import jax
import jax.numpy as jnp
from jax import lax
from jax.experimental import pallas as pl
from jax.experimental.pallas import tpu as pltpu
from jax.experimental.pallas import tpu_sc as plsc

_BINS = 15
_N = 16384
_C = 1000

_SC_N = 2048
_NW = 32
_RPW = _SC_N // _NW
_B = 16
_T = _RPW // _B

_R = 2048
_TC_OFF = _SC_N // _R

_EDGES = [
    0.0, 0.06666667014360428, 0.13333334028720856, 0.20000001788139343,
    0.2666666805744171, 0.3333333432674408, 0.40000003576278687,
    0.46666669845581055, 0.5333333611488342, 0.6000000238418579,
    0.6666666865348816, 0.7333333492279053, 0.8000000715255737,
    0.8666667342185974, 0.9333333969116211, 1.0,
]


def _sc_body(logits_hbm, labels_hbm, out_hbm, buf, lab, cnt, csum, asum, sem):
    w = lax.axis_index("s") * 2 + lax.axis_index("c")
    base = w * _RPW
    row_iota = lax.broadcasted_iota(jnp.int32, (_B,), 0)

    for b in range(_BINS):
        cnt[b, :] = jnp.zeros((_B,), jnp.float32)
        csum[b, :] = jnp.zeros((_B,), jnp.float32)
        asum[b, :] = jnp.zeros((_B,), jnp.float32)

    pltpu.async_copy(
        logits_hbm.at[pl.ds(base, _B), :], buf.at[0], sem.at[0])
    pltpu.async_copy(
        logits_hbm.at[pl.ds(base + _B, _B), :], buf.at[1], sem.at[1])

    neg_inf = jnp.full((_B,), -jnp.inf, jnp.float32)
    zeros = jnp.zeros((_B,), jnp.float32)

    def process(slot, row0):
        pltpu.make_async_copy(
            logits_hbm.at[pl.ds(0, _B), :], buf.at[slot], sem.at[slot]).wait()

        def jbody(c, carry):
            cvec, ms, ss = carry
            ms2, ss2 = list(ms), list(ss)
            for kk in range(8):
                v = plsc.load_gather(buf.at[slot], [row_iota, cvec + kk])
                k = kk % 4
                ms2[k] = jnp.maximum(ms2[k], v)
                ss2[k] = ss2[k] + jnp.exp(v)
            return cvec + 8, tuple(ms2), tuple(ss2)

        _, ms, ss = lax.fori_loop(
            0, 123, jbody,
            (row_iota, (neg_inf,) * 4, (zeros,) * 4))
        ms2, ss2 = list(ms), list(ss)
        for kk in range(16):
            raw = row_iota + (984 + kk)
            col = jnp.where(raw >= _C, raw - _C, raw)
            v = plsc.load_gather(buf.at[slot], [row_iota, col])
            k = kk % 4
            ms2[k] = jnp.maximum(ms2[k], v)
            ss2[k] = ss2[k] + jnp.exp(v)
        ms, ss = ms2, ss2
        m = jnp.maximum(jnp.maximum(ms[0], ms[1]), jnp.maximum(ms[2], ms[3]))
        s = (ss[0] + ss[1]) + (ss[2] + ss[3])

        pltpu.sync_copy(labels_hbm.at[pl.ds(row0, _B)], lab)
        xl = plsc.load_gather(buf.at[slot], [row_iota, lab[...]])
        correct = jnp.where(xl == m, 1.0, 0.0).astype(jnp.float32)
        p = jnp.exp(m) / s

        bidx = jnp.zeros((_B,), jnp.int32)
        for e in _EDGES[1:]:
            bidx = bidx + jnp.where(p > e, 1, 0).astype(jnp.int32)
        for b in range(_BINS):
            mask = bidx == b
            cnt[b, :] = cnt[b, :] + jnp.where(mask, 1.0, 0.0)
            csum[b, :] = csum[b, :] + jnp.where(mask, p, 0.0)
            asum[b, :] = asum[b, :] + jnp.where(mask, correct, 0.0)

    def obody(g, carry):
        for slot in range(2):
            t = 2 * g + slot
            row0 = base + t * _B
            process(slot, row0)
            nxt = t + 2

            @pl.when(nxt < _T)
            def _():
                pltpu.async_copy(
                    logits_hbm.at[pl.ds(base + nxt * _B, _B), :],
                    buf.at[slot], sem.at[slot])
        return carry

    lax.fori_loop(0, _T // 2, obody, 0)

    pltpu.sync_copy(cnt, out_hbm.at[w, 0])
    pltpu.sync_copy(csum, out_hbm.at[w, 1])
    pltpu.sync_copy(asum, out_hbm.at[w, 2])


def _sc_partials(logits, labels):
    mesh = plsc.VectorSubcoreMesh(core_axis_name="c", subcore_axis_name="s")
    kern = functools.partial(
        pl.kernel,
        out_type=jax.ShapeDtypeStruct((_NW, 3, _BINS, _B), jnp.float32),
        mesh=mesh,
        scratch_types=[
            pltpu.VMEM((2, _B, _C), jnp.float32),
            pltpu.VMEM((_B,), jnp.int32),
            pltpu.VMEM((_BINS, _B), jnp.float32),
            pltpu.VMEM((_BINS, _B), jnp.float32),
            pltpu.VMEM((_BINS, _B), jnp.float32),
            pltpu.SemaphoreType.DMA((2,)),
        ],
        compiler_params=pltpu.CompilerParams(needs_layout_passes=False),
    )(_sc_body)
    return kern(logits, labels)


def _tc_block(logits_ref, labels_ref, cnt_ref, csum_ref, asum_ref):
    step = pl.program_id(0)

    @pl.when(step == 0)
    def _init():
        cnt_ref[...] = jnp.zeros_like(cnt_ref)
        csum_ref[...] = jnp.zeros_like(csum_ref)
        asum_ref[...] = jnp.zeros_like(asum_ref)

    x = logits_ref[...]
    labels = labels_ref[0, 0, :]

    m = jnp.max(x, axis=-1, keepdims=True)
    s = jnp.sum(jnp.exp(x - m), axis=-1)
    p = 1.0 / s
    col = jax.lax.broadcasted_iota(jnp.int32, x.shape, 1)
    pred = jnp.min(jnp.where(x == m, col, _C), axis=-1)
    correct = (pred == labels).astype(jnp.float32)

    lane = jax.lax.broadcasted_iota(jnp.int32, (1, 128), 1)
    lo_row = jnp.full((1, 128), 2.0, dtype=jnp.float32)
    hi_row = jnp.full((1, 128), 3.0, dtype=jnp.float32)
    for i in range(_BINS):
        lo_row = jnp.where(lane == i, _EDGES[i], lo_row)
        hi_row = jnp.where(lane == i, _EDGES[i + 1], hi_row)
    pd = p[:, None]
    onehot = ((pd > lo_row) & (pd <= hi_row)).astype(jnp.float32)

    cnt_ref[...] += jnp.sum(onehot, axis=0, keepdims=True)
    csum_ref[...] += jnp.sum(onehot * pd, axis=0, keepdims=True)
    asum_ref[...] += jnp.sum(onehot * correct[:, None], axis=0, keepdims=True)


def _combine(sc_ref, cnt_ref, csum_ref, asum_ref, ece_ref):
    red = jnp.sum(jnp.sum(sc_ref[...], axis=0), axis=-1)
    cnt = red[0:1, :] + cnt_ref[0:1, 0:_BINS]
    csum = red[1:2, :] + csum_ref[0:1, 0:_BINS]
    asum = red[2:3, :] + asum_ref[0:1, 0:_BINS]
    safe = jnp.where(cnt > 0, cnt, 1.0)
    e = jnp.where(cnt > 0, csum / safe - asum / safe, 0.0)
    ece_ref[...] = jnp.sum(jnp.abs(e) * (cnt / _N)).reshape(1, 1)


@jax.jit
def _ece(logits, labels):
    labels = labels.astype(jnp.int32)
    sc_out = _sc_partials(logits, labels)

    grid = (_N - _SC_N) // _R
    labels3 = labels.reshape(_N // _R, 1, _R)
    acc = jax.ShapeDtypeStruct((1, 128), jnp.float32)
    cnt, csum, asum = pl.pallas_call(
        _tc_block,
        grid=(grid,),
        in_specs=[
            pl.BlockSpec((_R, _C), lambda i: (i + _TC_OFF, 0)),
            pl.BlockSpec((1, 1, _R), lambda i: (i + _TC_OFF, 0, 0)),
        ],
        out_specs=[
            pl.BlockSpec((1, 128), lambda i: (0, 0)),
            pl.BlockSpec((1, 128), lambda i: (0, 0)),
            pl.BlockSpec((1, 128), lambda i: (0, 0)),
        ],
        out_shape=[acc, acc, acc],
    )(logits, labels3)

    ece = pl.pallas_call(
        _combine,
        out_shape=jax.ShapeDtypeStruct((1, 1), jnp.float32),
    )(sc_out, cnt, csum, asum)
    return ece[0, 0]


def kernel(logits, labels):
    return _ece(logits, labels)
